# Initial kernel scaffold; baseline (speedup 1.0000x reference)
import jax
import jax.numpy as jnp
from jax.experimental import pallas as pl


def kernel(token_ids, weight):
    raise NotImplementedError("write your pallas kernel here")


# speedup vs baseline: 1.1096x; 1.1096x over previous
import functools

import jax
import jax.numpy as jnp
from jax import lax
from jax.experimental import pallas as pl
from jax.experimental.pallas import tpu as pltpu
from jax.experimental.pallas import tpu_sc as plsc

_info = plsc.get_sparse_core_info()
_NC, _NS = _info.num_cores, _info.num_subcores
_NW = _NC * _NS

_CHUNK = 1280


def _make_gather(B, V, D):
    assert B % _NW == 0
    b_per_w = B // _NW
    assert b_per_w % _CHUNK == 0
    nchunks = b_per_w // _CHUNK
    mesh = plsc.VectorSubcoreMesh(core_axis_name="c", subcore_axis_name="s")

    @functools.partial(
        pl.kernel,
        mesh=mesh,
        out_type=jax.ShapeDtypeStruct((B, D), jnp.float32),
        scratch_types=[
            pltpu.VMEM((b_per_w,), jnp.int32),
            pltpu.VMEM((_CHUNK, D), jnp.float32),
            pltpu.VMEM((_CHUNK, D), jnp.float32),
            pltpu.SemaphoreType.DMA,
            pltpu.SemaphoreType.DMA,
        ],
        compiler_params=pltpu.CompilerParams(use_tc_tiling_on_sc=False),
    )
    def k(idx_hbm, table_hbm, out_hbm, idx_v, rows_a, rows_b, sem_a, sem_b):
        wid = lax.axis_index("s") * _NC + lax.axis_index("c")
        base = pl.multiple_of(wid * b_per_w, b_per_w)
        pltpu.sync_copy(idx_hbm.at[pl.ds(base, b_per_w)], idx_v)

        bufs = (rows_a, rows_b)
        sems = (sem_a, sem_b)
        copies = [None, None]

        def start(g):
            b = g % 2
            copies[b] = pltpu.async_copy(
                table_hbm.at[idx_v.at[pl.ds(g * _CHUNK, _CHUNK)]],
                bufs[b],
                sems[b],
            )

        start(0)
        for g in range(nchunks):
            b = g % 2
            copies[b].wait()
            if g + 1 < nchunks:
                start(g + 1)
            off = pl.multiple_of(base + g * _CHUNK, _CHUNK)
            pltpu.sync_copy(bufs[b], out_hbm.at[pl.ds(off, _CHUNK)])

    return k


def kernel(token_ids, weight):
    V, D = weight.shape
    idx = token_ids.reshape(-1).astype(jnp.int32)
    B = idx.shape[0]
    out = _make_gather(B, V, D)(idx, weight)
    return out.reshape(token_ids.shape + (D,))


# speedup vs baseline: 1.1115x; 1.0017x over previous
import functools

import jax
import jax.numpy as jnp
from jax import lax
from jax.experimental import pallas as pl
from jax.experimental.pallas import tpu as pltpu
from jax.experimental.pallas import tpu_sc as plsc

_info = plsc.get_sparse_core_info()
_NC, _NS = _info.num_cores, _info.num_subcores
_NW = _NC * _NS

_CHUNK = 640
_NBUF = 4
_LAG = 2


def _make_gather(B, V, D):
    assert B % _NW == 0
    b_per_w = B // _NW
    assert b_per_w % _CHUNK == 0
    nchunks = b_per_w // _CHUNK
    mesh = plsc.VectorSubcoreMesh(core_axis_name="c", subcore_axis_name="s")

    @functools.partial(
        pl.kernel,
        mesh=mesh,
        out_type=jax.ShapeDtypeStruct((B, D), jnp.float32),
        scratch_types=[
            pltpu.VMEM((b_per_w,), jnp.int32),
            [pltpu.VMEM((_CHUNK, D), jnp.float32) for _ in range(_NBUF)],
            [pltpu.SemaphoreType.DMA for _ in range(_NBUF)],
            [pltpu.SemaphoreType.DMA for _ in range(_NBUF)],
        ],
        compiler_params=pltpu.CompilerParams(use_tc_tiling_on_sc=False),
    )
    def k(idx_hbm, table_hbm, out_hbm, idx_v, bufs, gsems, wsems):
        wid = lax.axis_index("s") * _NC + lax.axis_index("c")
        base = pl.multiple_of(wid * b_per_w, b_per_w)
        pltpu.sync_copy(idx_hbm.at[pl.ds(base, b_per_w)], idx_v)

        gcopies = [None] * _NBUF
        wcopies = [None] * _NBUF

        for i in range(nchunks + _LAG):
            if i < nchunks:
                b = i % _NBUF
                if i >= _NBUF:
                    wcopies[b].wait()
                gcopies[b] = pltpu.async_copy(
                    table_hbm.at[idx_v.at[pl.ds(i * _CHUNK, _CHUNK)]],
                    bufs[b],
                    gsems[b],
                )
            j = i - _LAG
            if j >= 0:
                bj = j % _NBUF
                gcopies[bj].wait()
                off = pl.multiple_of(base + j * _CHUNK, _CHUNK)
                wcopies[bj] = pltpu.async_copy(
                    bufs[bj], out_hbm.at[pl.ds(off, _CHUNK)], wsems[bj]
                )
        for j in range(max(0, nchunks - _NBUF), nchunks):
            wcopies[j % _NBUF].wait()

    return k


def kernel(token_ids, weight):
    V, D = weight.shape
    idx = token_ids.reshape(-1).astype(jnp.int32)
    B = idx.shape[0]
    out = _make_gather(B, V, D)(idx, weight)
    return out.reshape(token_ids.shape + (D,))


# speedup vs baseline: 1.7889x; 1.6095x over previous
import functools

import jax
import jax.numpy as jnp
from jax import lax
from jax.experimental import pallas as pl
from jax.experimental.pallas import tpu as pltpu
from jax.experimental.pallas import tpu_sc as plsc

_info = plsc.get_sparse_core_info()
_NC, _NS = _info.num_cores, _info.num_subcores
_NW = _NC * _NS

_G = 32


def _make_lookup(R, T, V, D):
    assert R % _NW == 0
    r_per_w = R // _NW
    assert r_per_w % _G == 0
    ngroups = r_per_w // _G
    mesh = plsc.VectorSubcoreMesh(core_axis_name="c", subcore_axis_name="s")

    @functools.partial(
        pl.kernel,
        mesh=mesh,
        out_type=jax.ShapeDtypeStruct((R, T, D), jnp.float32),
        scratch_types=[
            [pltpu.VMEM((_G, T), jnp.int32) for _ in range(2)],
            [pltpu.VMEM((_G, T, D), jnp.float32) for _ in range(2)],
            [pltpu.SemaphoreType.DMA for _ in range(2)],
            [pltpu.SemaphoreType.DMA for _ in range(2)],
            [pltpu.SemaphoreType.DMA for _ in range(2)],
        ],
        compiler_params=pltpu.CompilerParams(use_tc_tiling_on_sc=False),
    )
    def k(idx_hbm, table_hbm, out_hbm, ibufs, rbufs, isems, gsems, wsems):
        wid = lax.axis_index("s") * _NC + lax.axis_index("c")
        rbase = pl.multiple_of(wid * r_per_w, r_per_w)

        icopies = [None, None]
        gcopies = [[None] * _G, [None] * _G]
        wcopies = [None, None]

        def start_idx(g):
            b = g % 2
            icopies[b] = pltpu.async_copy(
                idx_hbm.at[pl.ds(rbase + g * _G, _G)], ibufs[b], isems[b]
            )

        start_idx(0)
        for g in range(ngroups):
            b = g % 2
            if g >= 2:
                wcopies[b].wait()
            icopies[b].wait()
            for j in range(_G):
                gcopies[b][j] = pltpu.async_copy(
                    table_hbm.at[ibufs[b].at[j]], rbufs[b].at[j], gsems[b]
                )
            if g + 1 < ngroups:
                start_idx(g + 1)
            for j in range(_G):
                gcopies[b][j].wait()
            wcopies[b] = pltpu.async_copy(
                rbufs[b], out_hbm.at[pl.ds(rbase + g * _G, _G)], wsems[b]
            )
        wcopies[(ngroups - 2) % 2].wait()
        wcopies[(ngroups - 1) % 2].wait()

    return k


def kernel(token_ids, weight):
    V, D = weight.shape
    R, T = token_ids.shape
    idx = token_ids.astype(jnp.int32)
    return _make_lookup(R, T, V, D)(idx, weight)
